# Initial kernel scaffold; baseline (speedup 1.0000x reference)
import jax
import jax.numpy as jnp
from jax.experimental import pallas as pl


def kernel(pred_logits, pred_boxes, orig_target_sizes):
    raise NotImplementedError("write your pallas kernel here")


# speedup vs baseline: 1.3063x; 1.3063x over previous
import jax
import jax.numpy as jnp
from jax.experimental import pallas as pl

_NUM_CLASSES = 80
_K = 300
_G = 25
_L = 125
_C = 128
_NEG = float("-inf")


def _pp_kernel(logits_ref, boxes_ref, sizes_ref,
               labels_ref, boxes_out_ref, scores_ref):
    rowmax = jnp.max(logits_ref[...], axis=2)
    row_ids = jax.lax.broadcasted_iota(jnp.int32, (_G, _L), 0) * _L + \
        jax.lax.broadcasted_iota(jnp.int32, (_G, _L), 1)
    lane_ids = jax.lax.broadcasted_iota(jnp.int32, (1, _C), 1)
    out_pos = jax.lax.broadcasted_iota(jnp.int32, (1, _K), 1)

    def body(k, carry):
        rmax, vals, idxs = carry
        m = jnp.max(rmax)
        r = jnp.min(jnp.where(rmax == m, row_ids, jnp.int32(_G * _L)))
        g = r // _L
        l = r % _L
        row = logits_ref[pl.dslice(g, 1), pl.dslice(l, 1), :].reshape(1, _C)
        c = jnp.min(jnp.where(row == m, lane_ids, jnp.int32(_C)))
        new_row = jnp.where(lane_ids == c, _NEG, row)
        logits_ref[pl.dslice(g, 1), pl.dslice(l, 1), :] = (
            new_row.reshape(1, 1, _C))
        rmax = jnp.where(row_ids == r, jnp.max(new_row), rmax)
        vals = jnp.where(out_pos == k, m, vals)
        idxs = jnp.where(out_pos == k, r * _C + c, idxs)
        return rmax, vals, idxs

    init = (rowmax,
            jnp.full((1, _K), _NEG, jnp.float32),
            jnp.zeros((1, _K), jnp.int32))
    _, top_vals, top_idx = jax.lax.fori_loop(0, _K, body, init)

    scores_ref[...] = jax.nn.sigmoid(top_vals)
    labels_ref[...] = jnp.remainder(top_idx, _NUM_CLASSES)
    qidx = top_idx // _NUM_CLASSES

    n = boxes_ref.shape[0]
    q_ids = jax.lax.broadcasted_iota(jnp.int32, (n, _K), 0)
    onehot = (q_ids == qidx).astype(jnp.float32)
    gathered = jax.lax.dot_general(
        onehot, boxes_ref[...],
        dimension_numbers=(((0,), (0,)), ((), ())),
        preferred_element_type=jnp.float32,
    )
    cx = gathered[:, 0:1]
    cy = gathered[:, 1:2]
    hw = 0.5 * gathered[:, 2:3]
    hh = 0.5 * gathered[:, 3:4]
    xyxy = jnp.concatenate([cx - hw, cy - hh, cx + hw, cy + hh], axis=1)
    scale = jnp.concatenate([sizes_ref[...], sizes_ref[...]], axis=1)
    boxes_out_ref[...] = xyxy * scale


def kernel(pred_logits, pred_boxes, orig_target_sizes):
    b, n, c = pred_logits.shape
    logits4 = pred_logits.reshape(b, _G, _L, _C)
    sizes = orig_target_sizes.reshape(b, 1, 2)
    labels, boxes, scores = pl.pallas_call(
        _pp_kernel,
        grid=(b,),
        in_specs=[
            pl.BlockSpec((None, _G, _L, _C), lambda i: (i, 0, 0, 0)),
            pl.BlockSpec((None, n, 4), lambda i: (i, 0, 0)),
            pl.BlockSpec((None, 1, 2), lambda i: (i, 0, 0)),
        ],
        out_specs=[
            pl.BlockSpec((None, 1, _K), lambda i: (i, 0, 0)),
            pl.BlockSpec((None, _K, 4), lambda i: (i, 0, 0)),
            pl.BlockSpec((None, 1, _K), lambda i: (i, 0, 0)),
        ],
        out_shape=[
            jax.ShapeDtypeStruct((b, 1, _K), jnp.int32),
            jax.ShapeDtypeStruct((b, _K, 4), jnp.float32),
            jax.ShapeDtypeStruct((b, 1, _K), jnp.float32),
        ],
    )(logits4, pred_boxes, sizes)
    return (labels.reshape(b, _K), boxes, scores.reshape(b, _K))
